# Initial kernel scaffold; baseline (speedup 1.0000x reference)
import jax
import jax.numpy as jnp
from jax.experimental import pallas as pl


def kernel(x, go_edge_index, go_etypes, block_edge_index, rgcn_w, rgcn_self_w, rgcn_b, fc0_w, fc0_b, gcn1_w, gcn1_b, fc1_w, fc1_b):
    raise NotImplementedError("write your pallas kernel here")


# speedup vs baseline: 487.0853x; 487.0853x over previous
import functools

import jax
import jax.numpy as jnp
from jax import lax
from jax.experimental import pallas as pl
from jax.experimental.pallas import tpu as pltpu
from jax.experimental.pallas import tpu_sc as plsc

N = 128
N_GOS = 4096
N_TOTAL = N * N_GOS
E_GO = 16777216
NB_CLASSES = 1024

NC = 2
NS = 16
NW = NC * NS
EPW = E_GO // NW
CHUNK_ROWS = 64
CHUNK = CHUNK_ROWS * 128
NCHUNK = EPW // CHUNK
ROWS_PER_WORKER = EPW // 128
TPW = N_TOTAL // NS


def _edge_kernel(table_hbm, ei_hbm, et_hbm, w_hbm, out_hbm,
                 src_v, dst_v, et_v, val_v, msg_v, zero_v, w_v,
                 table_sh, agg_sh, sem):
    cid = lax.axis_index("c")
    sid = lax.axis_index("s")
    wid = sid * NC + cid

    pltpu.sync_copy(table_hbm.at[pl.ds(sid * TPW, TPW)],
                    table_sh.at[pl.ds(sid * TPW, TPW)])
    pltpu.sync_copy(w_hbm, w_v)

    def _z(i, carry):
        zero_v[pl.ds(i * 16, 16)] = jnp.zeros((16,), jnp.float32)
        return carry
    lax.fori_loop(0, CHUNK // 16, _z, 0)
    for j in range(TPW // CHUNK):
        pltpu.sync_copy(zero_v, agg_sh.at[pl.ds(sid * TPW + j * CHUNK, CHUNK)])

    plsc.subcore_barrier()

    def _chunk(k, carry):
        base = wid * EPW + k * CHUNK
        pltpu.sync_copy(ei_hbm.at[0, pl.ds(base, CHUNK)], src_v)
        pltpu.sync_copy(ei_hbm.at[1, pl.ds(base, CHUNK)], dst_v)
        pltpu.sync_copy(et_hbm.at[pl.ds(base, CHUNK)], et_v)

        def _xf(i, c2):
            sl = pl.ds(i * 16, 16)
            s16 = src_v[sl]
            src_v[sl] = ((s16 & 127) << 12) | (s16 >> 7)
            d16 = dst_v[sl]
            dst_v[sl] = ((d16 & 127) << 12) | (d16 >> 7)
            return c2
        lax.fori_loop(0, CHUNK // 16, _xf, 0)

        pltpu.async_copy(table_sh.at[src_v], val_v, sem).wait()

        def _mul(i, c2):
            sl = pl.ds(i * 16, 16)
            w16 = plsc.load_gather(w_v, [et_v[sl]])
            msg_v[sl] = val_v[sl] * w16
            return c2
        lax.fori_loop(0, CHUNK // 16, _mul, 0)

        pltpu.sync_copy(msg_v, agg_sh.at[dst_v], add=True)
        return carry

    lax.fori_loop(0, NCHUNK, _chunk, 0)

    plsc.subcore_barrier()
    pltpu.sync_copy(agg_sh.at[pl.ds(sid * TPW, TPW)], out_hbm.at[cid, sid])


@functools.cache
def _rgcn_edge_pass_fn():
  return functools.partial(
    pl.kernel,
    out_type=jax.ShapeDtypeStruct((NC, NS, TPW), jnp.float32),
    mesh=plsc.VectorSubcoreMesh(core_axis_name="c", subcore_axis_name="s",
                                num_cores=NC),
    compiler_params=pltpu.CompilerParams(needs_layout_passes=False),
    scratch_types=[
        pltpu.VMEM((CHUNK,), jnp.int32),
        pltpu.VMEM((CHUNK,), jnp.int32),
        pltpu.VMEM((CHUNK,), jnp.int32),
        pltpu.VMEM((CHUNK,), jnp.float32),
        pltpu.VMEM((CHUNK,), jnp.float32),
        pltpu.VMEM((CHUNK,), jnp.float32),
        pltpu.VMEM((128,), jnp.float32),
        pltpu.VMEM_SHARED((N_TOTAL,), jnp.float32),
        pltpu.VMEM_SHARED((N_TOTAL,), jnp.float32),
        pltpu.SemaphoreType.DMA,
    ],
  )(_edge_kernel)


def _dense_kernel(x_ref, aggp_ref, bsr_ref, bsc_ref, bdr_ref, sw_ref, rb_ref,
                  fc0w_ref, fc0b_ref, g1w_ref, g1b_ref, fc1w_ref, fc1b_ref,
                  out_ref):
    f32 = jnp.float32
    sw = sw_ref[0, 0]
    rb = rb_ref[0, 0]
    agg = aggp_ref[0] + aggp_ref[1]
    h2 = agg + x_ref[...] * sw + rb

    z = jnp.maximum(
        jnp.dot(h2, fc0w_ref[...], preferred_element_type=f32) + fc0b_ref[...],
        0.0)

    node_row = lax.broadcasted_iota(jnp.int32, (N, 4096), 0)
    m_src_t = (node_row == bsr_ref[...]).astype(f32)
    m_dst_t = (node_row == bdr_ref[...]).astype(f32)
    node_col = lax.broadcasted_iota(jnp.int32, (4096, N), 1)
    m_src_e = (bsc_ref[...] == node_col).astype(f32)
    out_deg = jnp.sum(m_src_t, axis=1, keepdims=True)
    in_deg = jnp.sum(m_dst_t, axis=1, keepdims=True)
    norm_s = lax.rsqrt(jnp.maximum(out_deg, 1.0))
    norm_d = lax.rsqrt(jnp.maximum(in_deg, 1.0))
    s_mat = jnp.dot(m_dst_t, m_src_e, preferred_element_type=f32)
    feat = z * norm_s
    agg2 = jnp.dot(s_mat, feat, preferred_element_type=f32) * norm_d

    hg = jnp.dot(agg2, g1w_ref[...], preferred_element_type=f32) + g1b_ref[...]
    logits = jnp.dot(hg, fc1w_ref[...], preferred_element_type=f32) + fc1b_ref[...]
    out_ref[...] = jax.nn.sigmoid(logits)


def kernel(x, go_edge_index, go_etypes, block_edge_index, rgcn_w, rgcn_self_w,
           rgcn_b, fc0_w, fc0_b, gcn1_w, gcn1_b, fc1_w, fc1_b):
    table = x.reshape(N_TOTAL)
    ei = go_edge_index
    et = go_etypes
    w_vec = jnp.pad(rgcn_w[:, 0, 0], (0, 128 - rgcn_w.shape[0]))

    agg_parts = _rgcn_edge_pass_fn()(table, ei, et, w_vec)
    aggp = agg_parts.reshape(NC, N, N_GOS)

    bs_row = block_edge_index[0:1, :]
    bs_col = block_edge_index[0].reshape(4096, 1)
    bd_row = block_edge_index[1:2, :]
    sw = rgcn_self_w.reshape(1, 1)
    rb = rgcn_b.reshape(1, 1)

    smem = pl.BlockSpec(memory_space=pltpu.SMEM)

    out = pl.pallas_call(
        _dense_kernel,
        out_shape=jax.ShapeDtypeStruct((N, NB_CLASSES), jnp.float32),
        in_specs=[
            pl.BlockSpec(memory_space=pltpu.VMEM),
            pl.BlockSpec(memory_space=pltpu.VMEM),
            pl.BlockSpec(memory_space=pltpu.VMEM),
            pl.BlockSpec(memory_space=pltpu.VMEM),
            pl.BlockSpec(memory_space=pltpu.VMEM),
            smem,
            smem,
            pl.BlockSpec(memory_space=pltpu.VMEM),
            pl.BlockSpec(memory_space=pltpu.VMEM),
            pl.BlockSpec(memory_space=pltpu.VMEM),
            pl.BlockSpec(memory_space=pltpu.VMEM),
            pl.BlockSpec(memory_space=pltpu.VMEM),
            pl.BlockSpec(memory_space=pltpu.VMEM),
        ],
        out_specs=pl.BlockSpec(memory_space=pltpu.VMEM),
    )(x, aggp, bs_row, bs_col, bd_row, sw, rb,
      fc0_w, fc0_b.reshape(1, -1), gcn1_w, gcn1_b.reshape(1, -1),
      fc1_w, fc1_b.reshape(1, -1))
    return out


# speedup vs baseline: 577.0024x; 1.1846x over previous
import functools

import jax
import jax.numpy as jnp
from jax import lax
from jax.experimental import pallas as pl
from jax.experimental.pallas import tpu as pltpu
from jax.experimental.pallas import tpu_sc as plsc

N = 128
N_GOS = 4096
N_TOTAL = N * N_GOS
E_GO = 16777216
NB_CLASSES = 1024

NC = 2
NS = 16
NW = NC * NS
EPW = E_GO // NW
CHUNK_ROWS = 64
CHUNK = CHUNK_ROWS * 128
NCHUNK = EPW // CHUNK
ROWS_PER_WORKER = EPW // 128
TPW = N_TOTAL // NS


def _edge_kernel(table_hbm, ei_hbm, et_hbm, w_hbm, out_hbm,
                 src_v, dst_v, et_v, val_v, zero_v, w_v,
                 table_sh, agg_sh, sem):
    cid = lax.axis_index("c")
    sid = lax.axis_index("s")
    wid = sid * NC + cid

    pltpu.sync_copy(table_hbm.at[pl.ds(sid * TPW, TPW)],
                    table_sh.at[pl.ds(sid * TPW, TPW)])
    pltpu.sync_copy(w_hbm, w_v)

    def _z(i, carry):
        zero_v[pl.ds(i * 16, 16)] = jnp.zeros((16,), jnp.float32)
        return carry
    lax.fori_loop(0, CHUNK // 16, _z, 0)
    for j in range(TPW // CHUNK):
        pltpu.sync_copy(zero_v, agg_sh.at[pl.ds(sid * TPW + j * CHUNK, CHUNK)])

    plsc.subcore_barrier()

    def _chunk(k, carry):
        base = wid * EPW + k * CHUNK
        pltpu.sync_copy(ei_hbm.at[0, pl.ds(base, CHUNK)], src_v)
        pltpu.sync_copy(ei_hbm.at[1, pl.ds(base, CHUNK)], dst_v)
        pltpu.sync_copy(et_hbm.at[pl.ds(base, CHUNK)], et_v)

        pltpu.async_copy(table_sh.at[src_v], val_v, sem).wait()

        def _mul(i, c2):
            sl = pl.ds(i * 16, 16)
            w16 = plsc.load_gather(w_v, [et_v[sl]])
            val_v[sl] = val_v[sl] * w16
            return c2
        lax.fori_loop(0, CHUNK // 16, _mul, 0)

        pltpu.sync_copy(val_v, agg_sh.at[dst_v], add=True)
        return carry

    lax.fori_loop(0, NCHUNK, _chunk, 0)

    plsc.subcore_barrier()
    pltpu.sync_copy(agg_sh.at[pl.ds(sid * TPW, TPW)], out_hbm.at[cid, sid])


@functools.cache
def _rgcn_edge_pass_fn():
  return functools.partial(
    pl.kernel,
    out_type=jax.ShapeDtypeStruct((NC, NS, TPW), jnp.float32),
    mesh=plsc.VectorSubcoreMesh(core_axis_name="c", subcore_axis_name="s",
                                num_cores=NC),
    compiler_params=pltpu.CompilerParams(needs_layout_passes=False),
    scratch_types=[
        pltpu.VMEM((CHUNK,), jnp.int32),
        pltpu.VMEM((CHUNK,), jnp.int32),
        pltpu.VMEM((CHUNK,), jnp.int32),
        pltpu.VMEM((CHUNK,), jnp.float32),
        pltpu.VMEM((CHUNK,), jnp.float32),
        pltpu.VMEM((128,), jnp.float32),
        pltpu.VMEM_SHARED((N_TOTAL,), jnp.float32),
        pltpu.VMEM_SHARED((N_TOTAL,), jnp.float32),
        pltpu.SemaphoreType.DMA,
    ],
  )(_edge_kernel)


def _dense_kernel(xt_ref, aggp_ref, bsr_ref, bsc_ref, bdr_ref, sw_ref, rb_ref,
                  fc0w_ref, fc0b_ref, g1w_ref, g1b_ref, fc1w_ref, fc1b_ref,
                  out_ref):
    f32 = jnp.float32
    sw = sw_ref[0, 0]
    rb = rb_ref[0, 0]
    h2t = aggp_ref[0] + aggp_ref[1] + xt_ref[...] * sw + rb
    h2 = jnp.transpose(h2t)

    z = jnp.maximum(
        jnp.dot(h2, fc0w_ref[...], preferred_element_type=f32) + fc0b_ref[...],
        0.0)

    node_row = lax.broadcasted_iota(jnp.int32, (N, 4096), 0)
    m_src_t = (node_row == bsr_ref[...]).astype(f32)
    m_dst_t = (node_row == bdr_ref[...]).astype(f32)
    node_col = lax.broadcasted_iota(jnp.int32, (4096, N), 1)
    m_src_e = (bsc_ref[...] == node_col).astype(f32)
    out_deg = jnp.sum(m_src_t, axis=1, keepdims=True)
    in_deg = jnp.sum(m_dst_t, axis=1, keepdims=True)
    norm_s = lax.rsqrt(jnp.maximum(out_deg, 1.0))
    norm_d = lax.rsqrt(jnp.maximum(in_deg, 1.0))
    s_mat = jnp.dot(m_dst_t, m_src_e, preferred_element_type=f32)
    feat = z * norm_s
    agg2 = jnp.dot(s_mat, feat, preferred_element_type=f32) * norm_d

    hg = jnp.dot(agg2, g1w_ref[...], preferred_element_type=f32) + g1b_ref[...]
    logits = jnp.dot(hg, fc1w_ref[...], preferred_element_type=f32) + fc1b_ref[...]
    out_ref[...] = jax.nn.sigmoid(logits)


def kernel(x, go_edge_index, go_etypes, block_edge_index, rgcn_w, rgcn_self_w,
           rgcn_b, fc0_w, fc0_b, gcn1_w, gcn1_b, fc1_w, fc1_b):
    xt = jnp.transpose(x)
    table = xt.reshape(N_TOTAL)
    ei = go_edge_index
    et = go_etypes
    w_vec = jnp.pad(rgcn_w[:, 0, 0], (0, 128 - rgcn_w.shape[0]))

    agg_parts = _rgcn_edge_pass_fn()(table, ei, et, w_vec)
    aggp = agg_parts.reshape(NC, N_GOS, N)

    bs_row = block_edge_index[0:1, :]
    bs_col = block_edge_index[0].reshape(4096, 1)
    bd_row = block_edge_index[1:2, :]
    sw = rgcn_self_w.reshape(1, 1)
    rb = rgcn_b.reshape(1, 1)

    smem = pl.BlockSpec(memory_space=pltpu.SMEM)

    out = pl.pallas_call(
        _dense_kernel,
        out_shape=jax.ShapeDtypeStruct((N, NB_CLASSES), jnp.float32),
        in_specs=[
            pl.BlockSpec(memory_space=pltpu.VMEM),
            pl.BlockSpec(memory_space=pltpu.VMEM),
            pl.BlockSpec(memory_space=pltpu.VMEM),
            pl.BlockSpec(memory_space=pltpu.VMEM),
            pl.BlockSpec(memory_space=pltpu.VMEM),
            smem,
            smem,
            pl.BlockSpec(memory_space=pltpu.VMEM),
            pl.BlockSpec(memory_space=pltpu.VMEM),
            pl.BlockSpec(memory_space=pltpu.VMEM),
            pl.BlockSpec(memory_space=pltpu.VMEM),
            pl.BlockSpec(memory_space=pltpu.VMEM),
            pl.BlockSpec(memory_space=pltpu.VMEM),
        ],
        out_specs=pl.BlockSpec(memory_space=pltpu.VMEM),
    )(xt, aggp, bs_row, bs_col, bd_row, sw, rb,
      fc0_w, fc0_b.reshape(1, -1), gcn1_w, gcn1_b.reshape(1, -1),
      fc1_w, fc1_b.reshape(1, -1))
    return out


# speedup vs baseline: 944.0440x; 1.6361x over previous
import functools

import jax
import jax.numpy as jnp
from jax import lax
from jax.experimental import pallas as pl
from jax.experimental.pallas import tpu as pltpu
from jax.experimental.pallas import tpu_sc as plsc

N = 128
N_GOS = 4096
N_TOTAL = N * N_GOS
E_GO = 16777216
NB_CLASSES = 1024

NC = 2
NS = 16
NW = NC * NS
EPW = E_GO // NW
CHUNK = 4096
NCHUNK = EPW // CHUNK
ROWS_PER_WORKER = EPW // 128
TPW = N_TOTAL // NS


def _edge_kernel(table_hbm, ei_hbm, et_hbm, w_hbm, out_hbm,
                 src0, dst0, et0, val0, src1, dst1, et1, val1,
                 w_v, table_sh, agg_sh,
                 semL0, semL1, semG, semS):
    cid = lax.axis_index("c")
    sid = lax.axis_index("s")
    wid = sid * NC + cid
    ebase = wid * EPW

    bufs = ((src0, dst0, et0, val0, semL0),
            (src1, dst1, et1, val1, semL1))

    pltpu.sync_copy(table_hbm.at[pl.ds(sid * TPW, TPW)],
                    table_sh.at[pl.ds(sid * TPW, TPW)])
    pltpu.sync_copy(w_hbm, w_v)

    def _z(i, carry):
        val0[pl.ds(i * 16, 16)] = jnp.zeros((16,), jnp.float32)
        return carry
    lax.fori_loop(0, CHUNK // 16, _z, 0)
    for j in range(TPW // CHUNK):
        pltpu.sync_copy(val0, agg_sh.at[pl.ds(sid * TPW + j * CHUNK, CHUNK)])

    plsc.subcore_barrier()

    def _lin(k, b, start):
        src_v, dst_v, et_v, _, semL = b
        base = ebase + k * CHUNK
        for c in (
            pltpu.make_async_copy(ei_hbm.at[0, pl.ds(base, CHUNK)], src_v, semL),
            pltpu.make_async_copy(ei_hbm.at[1, pl.ds(base, CHUNK)], dst_v, semL),
            pltpu.make_async_copy(et_hbm.at[pl.ds(base, CHUNK)], et_v, semL),
        ):
            c.start() if start else c.wait()

    _lin(0, bufs[0], True)

    def _outer(k2, carry):
        for b in range(2):
            k = k2 * 2 + b
            src_v, dst_v, et_v, val_v, _ = bufs[b]
            _, dst_o, _, val_o, _ = bufs[1 - b]
            _lin(k, bufs[b], False)
            pltpu.make_async_copy(table_sh.at[src_v], val_v, semG).start()
            @pl.when(k > 0)
            def _wait_prev_scatter():
                pltpu.make_async_copy(val_o, agg_sh.at[dst_o], semS).wait()
            _lin(jnp.minimum(k + 1, NCHUNK - 1), bufs[1 - b], True)
            pltpu.make_async_copy(table_sh.at[src_v], val_v, semG).wait()

            @plsc.parallel_loop(0, CHUNK, 16, unroll=4)
            def _mul(i):
                sl = pl.ds(i, 16)
                w16 = plsc.load_gather(w_v, [et_v[sl]])
                val_v[sl] = val_v[sl] * w16

            pltpu.async_copy(val_v, agg_sh.at[dst_v], semS, add=True)
        return carry

    lax.fori_loop(0, NCHUNK // 2, _outer, 0)

    pltpu.make_async_copy(bufs[1][3], agg_sh.at[bufs[1][1]], semS).wait()
    _lin(NCHUNK - 1, bufs[0], False)

    plsc.subcore_barrier()
    pltpu.sync_copy(agg_sh.at[pl.ds(sid * TPW, TPW)], out_hbm.at[cid, sid])


@functools.cache
def _rgcn_edge_pass_fn():
  return functools.partial(
    pl.kernel,
    out_type=jax.ShapeDtypeStruct((NC, NS, TPW), jnp.float32),
    mesh=plsc.VectorSubcoreMesh(core_axis_name="c", subcore_axis_name="s",
                                num_cores=NC),
    compiler_params=pltpu.CompilerParams(needs_layout_passes=False),
    scratch_types=[
        pltpu.VMEM((CHUNK,), jnp.int32),
        pltpu.VMEM((CHUNK,), jnp.int32),
        pltpu.VMEM((CHUNK,), jnp.int32),
        pltpu.VMEM((CHUNK,), jnp.float32),
        pltpu.VMEM((CHUNK,), jnp.int32),
        pltpu.VMEM((CHUNK,), jnp.int32),
        pltpu.VMEM((CHUNK,), jnp.int32),
        pltpu.VMEM((CHUNK,), jnp.float32),
        pltpu.VMEM((128,), jnp.float32),
        pltpu.VMEM_SHARED((N_TOTAL,), jnp.float32),
        pltpu.VMEM_SHARED((N_TOTAL,), jnp.float32),
        pltpu.SemaphoreType.DMA,
        pltpu.SemaphoreType.DMA,
        pltpu.SemaphoreType.DMA,
        pltpu.SemaphoreType.DMA,
    ],
  )(_edge_kernel)


def _dense_kernel(xt_ref, aggp_ref, bsr_ref, bsc_ref, bdr_ref, sw_ref, rb_ref,
                  fc0w_ref, fc0b_ref, g1w_ref, g1b_ref, fc1w_ref, fc1b_ref,
                  out_ref):
    f32 = jnp.float32
    sw = sw_ref[0, 0]
    rb = rb_ref[0, 0]
    h2t = aggp_ref[0] + aggp_ref[1] + xt_ref[...] * sw + rb
    h2 = jnp.transpose(h2t)

    z = jnp.maximum(
        jnp.dot(h2, fc0w_ref[...], preferred_element_type=f32) + fc0b_ref[...],
        0.0)

    node_row = lax.broadcasted_iota(jnp.int32, (N, 4096), 0)
    m_src_t = (node_row == bsr_ref[...]).astype(f32)
    m_dst_t = (node_row == bdr_ref[...]).astype(f32)
    node_col = lax.broadcasted_iota(jnp.int32, (4096, N), 1)
    m_src_e = (bsc_ref[...] == node_col).astype(f32)
    out_deg = jnp.sum(m_src_t, axis=1, keepdims=True)
    in_deg = jnp.sum(m_dst_t, axis=1, keepdims=True)
    norm_s = lax.rsqrt(jnp.maximum(out_deg, 1.0))
    norm_d = lax.rsqrt(jnp.maximum(in_deg, 1.0))
    s_mat = jnp.dot(m_dst_t, m_src_e, preferred_element_type=f32)
    feat = z * norm_s
    agg2 = jnp.dot(s_mat, feat, preferred_element_type=f32) * norm_d

    hg = jnp.dot(agg2, g1w_ref[...], preferred_element_type=f32) + g1b_ref[...]
    logits = jnp.dot(hg, fc1w_ref[...], preferred_element_type=f32) + fc1b_ref[...]
    out_ref[...] = jax.nn.sigmoid(logits)


def kernel(x, go_edge_index, go_etypes, block_edge_index, rgcn_w, rgcn_self_w,
           rgcn_b, fc0_w, fc0_b, gcn1_w, gcn1_b, fc1_w, fc1_b):
    xt = jnp.transpose(x)
    table = xt.reshape(N_TOTAL)
    ei = go_edge_index
    et = go_etypes
    w_vec = jnp.pad(rgcn_w[:, 0, 0], (0, 128 - rgcn_w.shape[0]))

    agg_parts = _rgcn_edge_pass_fn()(table, ei, et, w_vec)
    aggp = agg_parts.reshape(NC, N_GOS, N)

    bs_row = block_edge_index[0:1, :]
    bs_col = block_edge_index[0].reshape(4096, 1)
    bd_row = block_edge_index[1:2, :]
    sw = rgcn_self_w.reshape(1, 1)
    rb = rgcn_b.reshape(1, 1)

    smem = pl.BlockSpec(memory_space=pltpu.SMEM)

    out = pl.pallas_call(
        _dense_kernel,
        out_shape=jax.ShapeDtypeStruct((N, NB_CLASSES), jnp.float32),
        in_specs=[
            pl.BlockSpec(memory_space=pltpu.VMEM),
            pl.BlockSpec(memory_space=pltpu.VMEM),
            pl.BlockSpec(memory_space=pltpu.VMEM),
            pl.BlockSpec(memory_space=pltpu.VMEM),
            pl.BlockSpec(memory_space=pltpu.VMEM),
            smem,
            smem,
            pl.BlockSpec(memory_space=pltpu.VMEM),
            pl.BlockSpec(memory_space=pltpu.VMEM),
            pl.BlockSpec(memory_space=pltpu.VMEM),
            pl.BlockSpec(memory_space=pltpu.VMEM),
            pl.BlockSpec(memory_space=pltpu.VMEM),
            pl.BlockSpec(memory_space=pltpu.VMEM),
        ],
        out_specs=pl.BlockSpec(memory_space=pltpu.VMEM),
    )(xt, aggp, bs_row, bs_col, bd_row, sw, rb,
      fc0_w, fc0_b.reshape(1, -1), gcn1_w, gcn1_b.reshape(1, -1),
      fc1_w, fc1_b.reshape(1, -1))
    return out


# speedup vs baseline: 1029.6693x; 1.0907x over previous
import functools

import jax
import jax.numpy as jnp
from jax import lax
from jax.experimental import pallas as pl
from jax.experimental.pallas import tpu as pltpu
from jax.experimental.pallas import tpu_sc as plsc

N = 128
N_GOS = 4096
N_TOTAL = N * N_GOS
E_GO = 16777216
NB_CLASSES = 1024

NC = 2
NS = 16
NW = NC * NS
EPW = E_GO // NW
CHUNK = 4096
NCHUNK = EPW // CHUNK
ROWS_PER_WORKER = EPW // 128
TPW = N_TOTAL // NS


def _edge_kernel(table_hbm, ei_hbm, et_hbm, w_hbm, out_hbm,
                 src0, dst0, et0, val0, src1, dst1, et1, val1,
                 src2, dst2, et2, val2,
                 w_v, table_sh, agg_sh,
                 semL0, semL1, semL2, semG, semS):
    cid = lax.axis_index("c")
    sid = lax.axis_index("s")
    wid = sid * NC + cid
    ebase = wid * EPW

    bufs = ((src0, dst0, et0, val0, semL0),
            (src1, dst1, et1, val1, semL1),
            (src2, dst2, et2, val2, semL2))

    pltpu.sync_copy(table_hbm.at[pl.ds(sid * TPW, TPW)],
                    table_sh.at[pl.ds(sid * TPW, TPW)])
    pltpu.sync_copy(w_hbm, w_v)

    def _z(i, carry):
        val0[pl.ds(i * 16, 16)] = jnp.zeros((16,), jnp.float32)
        return carry
    lax.fori_loop(0, CHUNK // 16, _z, 0)
    for j in range(TPW // CHUNK):
        pltpu.sync_copy(val0, agg_sh.at[pl.ds(sid * TPW + j * CHUNK, CHUNK)])

    plsc.subcore_barrier()

    def _clamp(k):
        if isinstance(k, int):
            return min(k, NCHUNK - 1)
        return jnp.minimum(k, NCHUNK - 1)

    def _lin(k, b, start):
        src_v, dst_v, et_v, _, semL = b
        base = ebase + _clamp(k) * CHUNK
        for c in (
            pltpu.make_async_copy(ei_hbm.at[0, pl.ds(base, CHUNK)], src_v, semL),
            pltpu.make_async_copy(ei_hbm.at[1, pl.ds(base, CHUNK)], dst_v, semL),
            pltpu.make_async_copy(et_hbm.at[pl.ds(base, CHUNK)], et_v, semL),
        ):
            c.start() if start else c.wait()

    def _iter(k, cur, nxt, nn, first):
        src_v, dst_v, et_v, val_v, _ = cur
        pltpu.make_async_copy(table_sh.at[src_v], val_v, semG).wait()
        @plsc.parallel_loop(0, CHUNK, 16, unroll=4)
        def _mul(i):
            sl = pl.ds(i, 16)
            w16 = plsc.load_gather(w_v, [et_v[sl]])
            val_v[sl] = val_v[sl] * w16
        if not first:
            pltpu.make_async_copy(nn[3], agg_sh.at[nn[1]], semS).wait()
        pltpu.async_copy(val_v, agg_sh.at[dst_v], semS, add=True)
        _lin(k + 1, nxt, False)
        pltpu.make_async_copy(table_sh.at[nxt[0]], nxt[3], semG).start()
        _lin(k + 2, nn, True)

    _lin(0, bufs[0], True)
    _lin(0, bufs[0], False)
    pltpu.make_async_copy(table_sh.at[bufs[0][0]], bufs[0][3], semG).start()
    _lin(1, bufs[1], True)

    _iter(0, bufs[0], bufs[1], bufs[2], True)

    def _outer(k3, carry):
        for b in range(3):
            k = 1 + k3 * 3 + b
            _iter(k, bufs[(1 + b) % 3], bufs[(2 + b) % 3], bufs[b], False)
        return carry
    lax.fori_loop(0, (NCHUNK - 2) // 3, _outer, 0)

    _iter(NCHUNK - 1, bufs[(NCHUNK - 1) % 3], bufs[NCHUNK % 3],
          bufs[(NCHUNK + 1) % 3], False)

    last = bufs[(NCHUNK - 1) % 3]
    tailg = bufs[NCHUNK % 3]
    taill = bufs[(NCHUNK + 1) % 3]
    pltpu.make_async_copy(last[3], agg_sh.at[last[1]], semS).wait()
    pltpu.make_async_copy(table_sh.at[tailg[0]], tailg[3], semG).wait()
    _lin(NCHUNK - 1, taill, False)

    plsc.subcore_barrier()
    pltpu.sync_copy(agg_sh.at[pl.ds(sid * TPW, TPW)], out_hbm.at[cid, sid])


@functools.cache
def _rgcn_edge_pass_fn():
  return functools.partial(
    pl.kernel,
    out_type=jax.ShapeDtypeStruct((NC, NS, TPW), jnp.float32),
    mesh=plsc.VectorSubcoreMesh(core_axis_name="c", subcore_axis_name="s",
                                num_cores=NC),
    compiler_params=pltpu.CompilerParams(needs_layout_passes=False),
    scratch_types=[
        pltpu.VMEM((CHUNK,), jnp.int32),
        pltpu.VMEM((CHUNK,), jnp.int32),
        pltpu.VMEM((CHUNK,), jnp.int32),
        pltpu.VMEM((CHUNK,), jnp.float32),
        pltpu.VMEM((CHUNK,), jnp.int32),
        pltpu.VMEM((CHUNK,), jnp.int32),
        pltpu.VMEM((CHUNK,), jnp.int32),
        pltpu.VMEM((CHUNK,), jnp.float32),
        pltpu.VMEM((CHUNK,), jnp.int32),
        pltpu.VMEM((CHUNK,), jnp.int32),
        pltpu.VMEM((CHUNK,), jnp.int32),
        pltpu.VMEM((CHUNK,), jnp.float32),
        pltpu.VMEM((128,), jnp.float32),
        pltpu.VMEM_SHARED((N_TOTAL,), jnp.float32),
        pltpu.VMEM_SHARED((N_TOTAL,), jnp.float32),
        pltpu.SemaphoreType.DMA,
        pltpu.SemaphoreType.DMA,
        pltpu.SemaphoreType.DMA,
        pltpu.SemaphoreType.DMA,
        pltpu.SemaphoreType.DMA,
    ],
  )(_edge_kernel)


def _dense_kernel(xt_ref, aggp_ref, bsr_ref, bsc_ref, bdr_ref, sw_ref, rb_ref,
                  fc0w_ref, fc0b_ref, g1w_ref, g1b_ref, fc1w_ref, fc1b_ref,
                  out_ref):
    f32 = jnp.float32
    sw = sw_ref[0, 0]
    rb = rb_ref[0, 0]
    h2t = aggp_ref[0] + aggp_ref[1] + xt_ref[...] * sw + rb
    h2 = jnp.transpose(h2t)

    z = jnp.maximum(
        jnp.dot(h2, fc0w_ref[...], preferred_element_type=f32) + fc0b_ref[...],
        0.0)

    node_row = lax.broadcasted_iota(jnp.int32, (N, 4096), 0)
    m_src_t = (node_row == bsr_ref[...]).astype(f32)
    m_dst_t = (node_row == bdr_ref[...]).astype(f32)
    node_col = lax.broadcasted_iota(jnp.int32, (4096, N), 1)
    m_src_e = (bsc_ref[...] == node_col).astype(f32)
    out_deg = jnp.sum(m_src_t, axis=1, keepdims=True)
    in_deg = jnp.sum(m_dst_t, axis=1, keepdims=True)
    norm_s = lax.rsqrt(jnp.maximum(out_deg, 1.0))
    norm_d = lax.rsqrt(jnp.maximum(in_deg, 1.0))
    s_mat = jnp.dot(m_dst_t, m_src_e, preferred_element_type=f32)
    feat = z * norm_s
    agg2 = jnp.dot(s_mat, feat, preferred_element_type=f32) * norm_d

    hg = jnp.dot(agg2, g1w_ref[...], preferred_element_type=f32) + g1b_ref[...]
    logits = jnp.dot(hg, fc1w_ref[...], preferred_element_type=f32) + fc1b_ref[...]
    out_ref[...] = jax.nn.sigmoid(logits)


def kernel(x, go_edge_index, go_etypes, block_edge_index, rgcn_w, rgcn_self_w,
           rgcn_b, fc0_w, fc0_b, gcn1_w, gcn1_b, fc1_w, fc1_b):
    xt = jnp.transpose(x)
    table = xt.reshape(N_TOTAL)
    ei = go_edge_index
    et = go_etypes
    w_vec = jnp.pad(rgcn_w[:, 0, 0], (0, 128 - rgcn_w.shape[0]))

    agg_parts = _rgcn_edge_pass_fn()(table, ei, et, w_vec)
    aggp = agg_parts.reshape(NC, N_GOS, N)

    bs_row = block_edge_index[0:1, :]
    bs_col = block_edge_index[0].reshape(4096, 1)
    bd_row = block_edge_index[1:2, :]
    sw = rgcn_self_w.reshape(1, 1)
    rb = rgcn_b.reshape(1, 1)

    smem = pl.BlockSpec(memory_space=pltpu.SMEM)

    out = pl.pallas_call(
        _dense_kernel,
        out_shape=jax.ShapeDtypeStruct((N, NB_CLASSES), jnp.float32),
        in_specs=[
            pl.BlockSpec(memory_space=pltpu.VMEM),
            pl.BlockSpec(memory_space=pltpu.VMEM),
            pl.BlockSpec(memory_space=pltpu.VMEM),
            pl.BlockSpec(memory_space=pltpu.VMEM),
            pl.BlockSpec(memory_space=pltpu.VMEM),
            smem,
            smem,
            pl.BlockSpec(memory_space=pltpu.VMEM),
            pl.BlockSpec(memory_space=pltpu.VMEM),
            pl.BlockSpec(memory_space=pltpu.VMEM),
            pl.BlockSpec(memory_space=pltpu.VMEM),
            pl.BlockSpec(memory_space=pltpu.VMEM),
            pl.BlockSpec(memory_space=pltpu.VMEM),
        ],
        out_specs=pl.BlockSpec(memory_space=pltpu.VMEM),
    )(xt, aggp, bs_row, bs_col, bd_row, sw, rb,
      fc0_w, fc0_b.reshape(1, -1), gcn1_w, gcn1_b.reshape(1, -1),
      fc1_w, fc1_b.reshape(1, -1))
    return out
